# baseline (speedup 1.0000x reference)
import math

import numpy as np
import jax
import jax.numpy as jnp
from jax import lax
from jax.experimental import pallas as pl
from jax.experimental.pallas import tpu as pltpu

_H = 6
_WS = 8
_N = _WS * _WS
_P = _N * _N
_T = (2 * _WS - 1) ** 2


def _build_tables():
    ch = np.arange(-(_WS - 1), _WS, dtype=np.float32)
    t = np.stack(np.meshgrid(ch, ch, indexing="ij"), axis=-1)
    t /= float(_WS - 1)
    t *= 8.0
    t = np.sign(t) * np.log2(np.abs(t) + 1.0) / np.log2(8.0)
    coords_t = t.reshape(_T, 2).T.copy()

    c = np.arange(_WS)
    grid = np.stack(np.meshgrid(c, c, indexing="ij")).reshape(2, -1)
    rel = (grid[:, :, None] - grid[:, None, :]).transpose(1, 2, 0)
    rel = rel.astype(np.int64)
    rel[:, :, 0] += _WS - 1
    rel[:, :, 1] += _WS - 1
    rel[:, :, 0] *= 2 * _WS - 1
    idx = rel.sum(-1).reshape(-1)

    onehot_t = np.zeros((_T, _P), dtype=np.float32)
    onehot_t[idx, np.arange(_P)] = 1.0
    return coords_t, onehot_t


_TT_NP, _OT_NP = _build_tables()


def _bias_kernel(ls_ref, w1_ref, b1_ref, w2_ref, tt_ref, ot_ref,
                 bias_ref, scale_ref):
    h = lax.dot_general(w1_ref[...], tt_ref[...], (((0,), (0,)), ((), ())),
                        preferred_element_type=jnp.float32)
    h = jnp.maximum(h + b1_ref[...], 0.0)
    bt = lax.dot_general(w2_ref[...], h, (((0,), (0,)), ((), ())),
                         preferred_element_type=jnp.float32)
    bias = jnp.dot(bt, ot_ref[...], preferred_element_type=jnp.float32)
    bias_ref[...] = 16.0 * jax.nn.sigmoid(bias)
    scale_ref[...] = jnp.exp(jnp.minimum(ls_ref[...], math.log(100.0)))


def _apply_kernel(attn_ref, scale_ref, bias_ref, out_ref):
    out_ref[...] = attn_ref[...] * scale_ref[...][None] + bias_ref[...][None]


def kernel(attn, x_size, logit_scale, w1, b1, w2):
    del x_size
    B = attn.shape[0]
    attn3 = attn.reshape(B, _H, _P)

    tt = jnp.asarray(_TT_NP)
    ot = jnp.asarray(_OT_NP)
    ls2 = logit_scale.reshape(_H, 1)
    b1c = b1.reshape(-1, 1)

    bias, scale = pl.pallas_call(
        _bias_kernel,
        out_shape=(
            jax.ShapeDtypeStruct((_H, _P), jnp.float32),
            jax.ShapeDtypeStruct((_H, 1), jnp.float32),
        ),
    )(ls2, w1, b1c, w2, tt, ot)

    BB = 8
    out3 = pl.pallas_call(
        _apply_kernel,
        grid=(B // BB,),
        in_specs=[
            pl.BlockSpec((BB, _H, _P), lambda i: (i, 0, 0)),
            pl.BlockSpec((_H, 1), lambda i: (0, 0)),
            pl.BlockSpec((_H, _P), lambda i: (0, 0)),
        ],
        out_specs=pl.BlockSpec((BB, _H, _P), lambda i: (i, 0, 0)),
        out_shape=jax.ShapeDtypeStruct((B, _H, _P), jnp.float32),
        compiler_params=pltpu.CompilerParams(
            dimension_semantics=("arbitrary",),
        ),
    )(attn3, scale, bias)
    return out3.reshape(attn.shape)
